# baseline (speedup 1.0000x reference)
import functools
import jax
import jax.numpy as jnp
from jax import lax
from jax.experimental import pallas as pl
from jax.experimental.pallas import tpu as pltpu
from jax.experimental.pallas import tpu_sc as plsc

N = 50000
E = 800000
NT = 32

PG = E // NT
CG = 1024
NCG = 24
TAILG = PG - NCG * CG

BE = 6400
NBE = E // BE

E_R = 802816
PS = E_R // NT
NCOL = 22
SCHUNK = 512
NSC = PS // SCHUNK
NSTR = SCHUNK // 128
N_ACC = 50048
ZROWS = 3128
OROWS = 3128

BN = 2000
NBN = N // BN


@functools.cache
def _make_gather():
    mesh = plsc.VectorSubcoreMesh(core_axis_name="c", subcore_axis_name="s")

    @functools.partial(
        pl.kernel,
        out_type=[jax.ShapeDtypeStruct((E,), jnp.float32) for _ in range(4)],
        mesh=mesh,
        compiler_params=pltpu.CompilerParams(needs_layout_passes=False),
        scratch_types=[
            pltpu.VMEM((2 * N,), jnp.float32),
            pltpu.VMEM((CG,), jnp.int32),
            pltpu.VMEM((CG,), jnp.int32),
            pltpu.VMEM((CG,), jnp.float32),
            pltpu.VMEM((CG,), jnp.float32),
            pltpu.VMEM((CG,), jnp.float32),
            pltpu.VMEM((CG,), jnp.float32),
        ],
    )
    def gather_kernel(xflat_hbm, src_hbm, dst_hbm,
                      o_s0, o_s1, o_d0, o_d1,
                      xv, isv, idv, b_s0, b_s1, b_d0, b_d1):
        wid = lax.axis_index("s") * 2 + lax.axis_index("c")
        base = wid * PG
        pltpu.sync_copy(xflat_hbm, xv)

        def gather16(i, clamp):
            s = isv[pl.ds(i * 16, 16)]
            d = idv[pl.ds(i * 16, 16)]
            if clamp:
                s = jnp.clip(s, 0, N - 1)
                d = jnp.clip(d, 0, N - 1)
            s = s * 2
            d = d * 2
            b_s0[pl.ds(i * 16, 16)] = plsc.load_gather(xv, [s])
            b_s1[pl.ds(i * 16, 16)] = plsc.load_gather(xv, [s + 1])
            b_d0[pl.ds(i * 16, 16)] = plsc.load_gather(xv, [d])
            b_d1[pl.ds(i * 16, 16)] = plsc.load_gather(xv, [d + 1])

        def run_chunk(off, n_edges, n_full16, tail16):
            pltpu.sync_copy(src_hbm.at[pl.ds(off, n_edges)],
                            isv.at[pl.ds(0, n_edges)])
            pltpu.sync_copy(dst_hbm.at[pl.ds(off, n_edges)],
                            idv.at[pl.ds(0, n_edges)])

            def body(i, carry):
                gather16(i, False)
                return carry

            lax.fori_loop(0, n_full16, body, None)
            if tail16:
                gather16(n_full16, True)
            pltpu.sync_copy(b_s0.at[pl.ds(0, n_edges)],
                            o_s0.at[pl.ds(off, n_edges)])
            pltpu.sync_copy(b_s1.at[pl.ds(0, n_edges)],
                            o_s1.at[pl.ds(off, n_edges)])
            pltpu.sync_copy(b_d0.at[pl.ds(0, n_edges)],
                            o_d0.at[pl.ds(off, n_edges)])
            pltpu.sync_copy(b_d1.at[pl.ds(0, n_edges)],
                            o_d1.at[pl.ds(off, n_edges)])

        for c in range(NCG):
            run_chunk(base + c * CG, CG, CG // 16, False)
        run_chunk(base + NCG * CG, TAILG, TAILG // 16, True)

    return gather_kernel


def _edge_body(c0_ref, dt_ref, c2_ref, lane_ref, s0_ref, s1_ref, d0_ref,
               d1_ref, tew_ref, telam_ref, sw_ref, sb_ref, wat_ref,
               batt_ref, va_ref, *o_refs):
    row = lambda ref: ref[0, 0, :].reshape(1, BE)
    c0 = row(c0_ref)
    dt = row(dt_ref)
    c2 = row(c2_ref)
    lane = row(lane_ref)
    oh = (lane == lax.broadcasted_iota(jnp.int32, (8, BE), 0)
          ).astype(jnp.float32)
    cdot = lambda a, b: lax.dot_general(a, b, (((0,), (0,)), ((), ())))
    ret_w = cdot(tew_ref[...], oh)
    ret_lam = cdot(telam_ref[...], oh)
    arg = dt * ret_w
    sh = sw_ref[...] * dt + sb_ref[...]
    lam = jnp.exp(-jnp.square(ret_lam))
    te_s = (1.0 - lam) * jnp.sin(arg) + lam * jnp.sin(sh)
    te_c = (1.0 - lam) * jnp.cos(arg) + lam * jnp.cos(sh)
    s0 = row(s0_ref)
    s1 = row(s1_ref)
    d0 = row(d0_ref)
    d1 = row(d1_ref)
    wat = wat_ref[...]
    col = lambda j: wat[:, j:j + 1]
    rdot = lambda a, b: lax.dot_general(a, b, (((1,), (0,)), ((), ())))
    z = (col(0) * s0 + col(1) * s1 + col(2) * d0 + col(3) * d1
         + col(4) * c0 + col(5) * dt + col(6) * c2
         + rdot(wat[:, 7:15], te_s)
         + rdot(wat[:, 15:23], te_c)
         + batt_ref[...])
    va = va_ref[...]
    att = rdot(va, jnp.tanh(z))
    bound = jnp.sum(jnp.abs(va))
    ex = jnp.exp(att - bound)
    es = ex * te_s
    ec = ex * te_c
    cols = [ex, ex * s0, ex * s1, ex * c0, ex * dt, ex * c2]
    cols += [es[j:j + 1, :] for j in range(8)]
    cols += [ec[j:j + 1, :] for j in range(8)]
    for j in range(NCOL):
        o_refs[j][0, 0, :] = cols[j].reshape(BE)


def _edge_call(c0, dtc, c2, lane3, s03, s13, d03, d13,
               tew, telam, sw2, sb2, wat, batt2, va):
    flat3 = pl.BlockSpec((1, 1, BE), lambda i: (0, 0, i))
    full = lambda a: pl.BlockSpec(a.shape, lambda i: tuple(0 for _ in a.shape))
    return pl.pallas_call(
        _edge_body,
        grid=(NBE,),
        in_specs=[
            flat3, flat3, flat3, flat3, flat3, flat3, flat3, flat3,
            full(tew), full(telam), full(sw2), full(sb2),
            full(wat), full(batt2), full(va),
        ],
        out_specs=[pl.BlockSpec((1, 1, BE), lambda i: (0, 0, i))
                   for _ in range(NCOL)],
        out_shape=[jax.ShapeDtypeStruct((1, 1, E_R), jnp.float32)
                   for _ in range(NCOL)],
    )(c0, dtc, c2, lane3, s03, s13, d03, d13,
      tew, telam, sw2, sb2, wat, batt2, va)


@functools.cache
def _make_scatter():
    mesh = plsc.VectorSubcoreMesh(core_axis_name="c", subcore_axis_name="s")

    @functools.partial(
        pl.kernel,
        out_type=jax.ShapeDtypeStruct((2 * N, 24), jnp.float32),
        mesh=mesh,
        compiler_params=pltpu.CompilerParams(
            needs_layout_passes=False, use_tc_tiling_on_sc=False),
        scratch_types=[
            pltpu.VMEM_SHARED((N_ACC, 24), jnp.float32),
            pltpu.VMEM((PS // 128, 128), jnp.int32),
            pltpu.VMEM((NCOL, SCHUNK), jnp.float32),
            pltpu.VMEM((SCHUNK, 24), jnp.float32),
        ],
    )
    def scatter_kernel(*args):
        col_hbm = args[:NCOL]
        dst3_hbm, zero_hbm, out_hbm, acc, idxv, cols, rows = args[NCOL:]
        cid = lax.axis_index("c")
        sid = lax.axis_index("s")
        wid = sid * 2 + cid
        base = wid * PS
        pltpu.sync_copy(zero_hbm,
                        acc.at[pl.ds(pl.multiple_of(sid * ZROWS, 8), ZROWS)])
        pltpu.sync_copy(dst3_hbm.at[wid], idxv)
        pltpu.sync_copy(zero_hbm.at[pl.ds(0, SCHUNK)], rows)
        plsc.subcore_barrier()
        lane16 = lax.iota(jnp.int32, 16)
        jvecs = [jnp.full((16,), j, jnp.int32) for j in range(NCOL)]

        def chunk(c, carry):
            r_off = pl.multiple_of(base + c * SCHUNK, 8)
            for j in range(NCOL):
                pltpu.sync_copy(col_hbm[j].at[pl.ds(r_off, SCHUNK)],
                                cols.at[j])

            def interleave(g, carry2):
                ridx = g * 16 + lane16
                for j in range(NCOL):
                    vals = cols[j, pl.ds(g * 16, 16)]
                    plsc.store_scatter(rows, [ridx, jvecs[j]], vals)
                return carry2

            lax.fori_loop(0, SCHUNK // 16, interleave, None)
            for k in range(NSTR):
                pltpu.sync_copy(rows.at[pl.ds(k * 128, 128)],
                                acc.at[idxv.at[c * NSTR + k]], add=True)
            return carry

        lax.fori_loop(0, NSC, chunk, None)
        plsc.subcore_barrier()
        a_off = pl.multiple_of(sid * OROWS, 8)
        o_off = pl.multiple_of(cid * N + sid * OROWS, 8)

        @pl.when(sid < 15)
        def _():
            pltpu.sync_copy(acc.at[pl.ds(a_off, OROWS)],
                            out_hbm.at[pl.ds(o_off, OROWS)])

        @pl.when(sid == 15)
        def _():
            pltpu.sync_copy(acc.at[pl.ds(a_off, N - 15 * OROWS)],
                            out_hbm.at[pl.ds(o_off, N - 15 * OROWS)])

    return scatter_kernel


def _mlp_body(p0_ref, p1_ref, w1_ref, b1_ref, w2_ref, b2_ref,
              w3_ref, b3_ref, o_ref):
    s = p0_ref[...] + p1_ref[...]
    den = s[:, 0:1]
    den = jnp.where(den == 0.0, 1.0, den)
    h = s[:, 1:22] / den
    h = jnp.maximum(jnp.dot(h, w1_ref[...]) + b1_ref[...], 0.0)
    h = jnp.maximum(jnp.dot(h, w2_ref[...]) + b2_ref[...], 0.0)
    h = jnp.maximum(jnp.dot(h, w3_ref[...]) + b3_ref[...], 0.0)
    o_ref[...] = h


def _mlp_call(partials, w1t, b1r, w2t, b2r, w3t, b3r):
    full = lambda a: pl.BlockSpec(a.shape, lambda i: tuple(0 for _ in a.shape))
    return pl.pallas_call(
        _mlp_body,
        grid=(NBN,),
        in_specs=[
            pl.BlockSpec((BN, 24), lambda i: (i, 0)),
            pl.BlockSpec((BN, 24), lambda i: (i + NBN, 0)),
            full(w1t), full(b1r), full(w2t), full(b2r), full(w3t), full(b3r),
        ],
        out_specs=pl.BlockSpec((BN, 32), lambda i: (i, 0)),
        out_shape=jax.ShapeDtypeStruct((N, 32), jnp.float32),
    )(partials, partials, w1t, b1r, w2t, b2r, w3t, b3r)


def kernel(X_msg, edge_feat3, edge_lane, edge_src, edge_dst,
           TE_w, TE_lam, shared_W, shared_b,
           W_att, b_att, va, W1, b1, W2, b2, W3, b3):
    src_i = edge_src.astype(jnp.int32)
    dst_i = edge_dst.astype(jnp.int32)
    xflat = X_msg.reshape(-1)

    s0, s1, d0, d1 = _make_gather()(xflat, src_i, dst_i)

    e3 = lambda a: a.reshape(1, 1, E)
    r_cols = _edge_call(
        e3(edge_feat3[:, 0]), e3(edge_feat3[:, 1]), e3(edge_feat3[:, 2]),
        e3(edge_lane.astype(jnp.int32)),
        e3(s0), e3(s1), e3(d0), e3(d1),
        TE_w, TE_lam,
        shared_W, shared_b.reshape(8, 1),
        W_att, b_att.reshape(32, 1), va,
    )

    dst3 = jnp.pad(dst_i, (0, E_R - E),
                   constant_values=N_ACC - 1).reshape(NT, PS // 128, 128)
    partials = _make_scatter()(
        *[c.reshape(E_R) for c in r_cols],
        dst3, jnp.zeros((ZROWS, 24), jnp.float32))

    return _mlp_call(
        partials,
        W1.T, b1.reshape(1, 32),
        W2.T, b2.reshape(1, 32),
        W3.T, b3.reshape(1, 32),
    )


# speedup vs baseline: 1.5948x; 1.5948x over previous
import functools
import jax
import jax.numpy as jnp
from jax import lax
from jax.experimental import pallas as pl
from jax.experimental.pallas import tpu as pltpu
from jax.experimental.pallas import tpu_sc as plsc

N = 50000
E = 800000
NT = 32

PG = E // NT
CG = 1024
NCG = 24
TAILG = PG - NCG * CG

BE = 6400
NBE = E // BE

E_R = 802816
PS = E_R // NT
NCOL = 22
SCHUNK = 512
NSC = PS // SCHUNK
NSTR = SCHUNK // 128
N_ACC = 50048
ZROWS = 3128
OROWS = 3128

BN = 2000
NBN = N // BN


@functools.cache
def _make_gather():
    mesh = plsc.VectorSubcoreMesh(core_axis_name="c", subcore_axis_name="s")

    @functools.partial(
        pl.kernel,
        out_type=[jax.ShapeDtypeStruct((E,), jnp.float32) for _ in range(4)],
        mesh=mesh,
        compiler_params=pltpu.CompilerParams(needs_layout_passes=False),
        scratch_types=[
            pltpu.VMEM((2 * N,), jnp.float32),
            pltpu.VMEM((CG,), jnp.int32),
            pltpu.VMEM((CG,), jnp.int32),
            pltpu.VMEM((CG,), jnp.float32),
            pltpu.VMEM((CG,), jnp.float32),
            pltpu.VMEM((CG,), jnp.float32),
            pltpu.VMEM((CG,), jnp.float32),
        ],
    )
    def gather_kernel(xflat_hbm, src_hbm, dst_hbm,
                      o_s0, o_s1, o_d0, o_d1,
                      xv, isv, idv, b_s0, b_s1, b_d0, b_d1):
        wid = lax.axis_index("s") * 2 + lax.axis_index("c")
        base = wid * PG
        pltpu.sync_copy(xflat_hbm, xv)

        def gather16(i, clamp):
            s = isv[pl.ds(i * 16, 16)]
            d = idv[pl.ds(i * 16, 16)]
            if clamp:
                s = jnp.clip(s, 0, N - 1)
                d = jnp.clip(d, 0, N - 1)
            s = s * 2
            d = d * 2
            b_s0[pl.ds(i * 16, 16)] = plsc.load_gather(xv, [s])
            b_s1[pl.ds(i * 16, 16)] = plsc.load_gather(xv, [s + 1])
            b_d0[pl.ds(i * 16, 16)] = plsc.load_gather(xv, [d])
            b_d1[pl.ds(i * 16, 16)] = plsc.load_gather(xv, [d + 1])

        def run_chunk(off, n_edges, n_full16, tail16):
            pltpu.sync_copy(src_hbm.at[pl.ds(off, n_edges)],
                            isv.at[pl.ds(0, n_edges)])
            pltpu.sync_copy(dst_hbm.at[pl.ds(off, n_edges)],
                            idv.at[pl.ds(0, n_edges)])

            def body(i, carry):
                gather16(i, False)
                return carry

            lax.fori_loop(0, n_full16, body, None)
            if tail16:
                gather16(n_full16, True)
            pltpu.sync_copy(b_s0.at[pl.ds(0, n_edges)],
                            o_s0.at[pl.ds(off, n_edges)])
            pltpu.sync_copy(b_s1.at[pl.ds(0, n_edges)],
                            o_s1.at[pl.ds(off, n_edges)])
            pltpu.sync_copy(b_d0.at[pl.ds(0, n_edges)],
                            o_d0.at[pl.ds(off, n_edges)])
            pltpu.sync_copy(b_d1.at[pl.ds(0, n_edges)],
                            o_d1.at[pl.ds(off, n_edges)])

        for c in range(NCG):
            run_chunk(base + c * CG, CG, CG // 16, False)
        run_chunk(base + NCG * CG, TAILG, TAILG // 16, True)

    return gather_kernel


def _edge_body(c0_ref, dt_ref, c2_ref, lane_ref, s0_ref, s1_ref, d0_ref,
               d1_ref, tew_ref, telam_ref, sw_ref, sb_ref, wat_ref,
               batt_ref, va_ref, *o_refs):
    row = lambda ref: ref[0, 0, :].reshape(1, BE)
    c0 = row(c0_ref)
    dt = row(dt_ref)
    c2 = row(c2_ref)
    lane = row(lane_ref)
    oh = (lane == lax.broadcasted_iota(jnp.int32, (8, BE), 0)
          ).astype(jnp.float32)
    cdot = lambda a, b: lax.dot_general(a, b, (((0,), (0,)), ((), ())))
    ret_w = cdot(tew_ref[...], oh)
    ret_lam = cdot(telam_ref[...], oh)
    arg = dt * ret_w
    sh = sw_ref[...] * dt + sb_ref[...]
    lam = jnp.exp(-jnp.square(ret_lam))
    te_s = (1.0 - lam) * jnp.sin(arg) + lam * jnp.sin(sh)
    te_c = (1.0 - lam) * jnp.cos(arg) + lam * jnp.cos(sh)
    s0 = row(s0_ref)
    s1 = row(s1_ref)
    d0 = row(d0_ref)
    d1 = row(d1_ref)
    wat = wat_ref[...]
    col = lambda j: wat[:, j:j + 1]
    rdot = lambda a, b: lax.dot_general(a, b, (((1,), (0,)), ((), ())))
    z = (col(0) * s0 + col(1) * s1 + col(2) * d0 + col(3) * d1
         + col(4) * c0 + col(5) * dt + col(6) * c2
         + rdot(wat[:, 7:15], te_s)
         + rdot(wat[:, 15:23], te_c)
         + batt_ref[...])
    va = va_ref[...]
    att = rdot(va, jnp.tanh(z))
    bound = jnp.sum(jnp.abs(va))
    ex = jnp.exp(att - bound)
    es = ex * te_s
    ec = ex * te_c
    cols = [ex, ex * s0, ex * s1, ex * c0, ex * dt, ex * c2]
    cols += [es[j:j + 1, :] for j in range(8)]
    cols += [ec[j:j + 1, :] for j in range(8)]
    for j in range(NCOL):
        o_refs[j][0, 0, :] = cols[j].reshape(BE)


def _edge_call(c0, dtc, c2, lane3, s03, s13, d03, d13,
               tew, telam, sw2, sb2, wat, batt2, va):
    flat3 = pl.BlockSpec((1, 1, BE), lambda i: (0, 0, i))
    full = lambda a: pl.BlockSpec(a.shape, lambda i: tuple(0 for _ in a.shape))
    return pl.pallas_call(
        _edge_body,
        grid=(NBE,),
        in_specs=[
            flat3, flat3, flat3, flat3, flat3, flat3, flat3, flat3,
            full(tew), full(telam), full(sw2), full(sb2),
            full(wat), full(batt2), full(va),
        ],
        out_specs=[pl.BlockSpec((1, 1, BE), lambda i: (0, 0, i))
                   for _ in range(NCOL)],
        out_shape=[jax.ShapeDtypeStruct((1, 1, E_R), jnp.float32)
                   for _ in range(NCOL)],
    )(c0, dtc, c2, lane3, s03, s13, d03, d13,
      tew, telam, sw2, sb2, wat, batt2, va)


@functools.cache
def _make_scatter():
    mesh = plsc.VectorSubcoreMesh(core_axis_name="c", subcore_axis_name="s")

    @functools.partial(
        pl.kernel,
        out_type=jax.ShapeDtypeStruct((2 * N, 24), jnp.float32),
        mesh=mesh,
        compiler_params=pltpu.CompilerParams(
            needs_layout_passes=False, use_tc_tiling_on_sc=False),
        scratch_types=[
            pltpu.VMEM_SHARED((N_ACC, 24), jnp.float32),
            pltpu.VMEM((PS // 128, 128), jnp.int32),
            pltpu.VMEM((NCOL, SCHUNK), jnp.float32),
            pltpu.VMEM((SCHUNK, 24), jnp.float32),
            pltpu.SemaphoreType.DMA,
        ],
    )
    def scatter_kernel(*args):
        col_hbm = args[:NCOL]
        dst3_hbm, zero_hbm, out_hbm, acc, idxv, cols, rows, sem = args[NCOL:]
        cid = lax.axis_index("c")
        sid = lax.axis_index("s")
        wid = sid * 2 + cid
        base = wid * PS
        pltpu.sync_copy(zero_hbm,
                        acc.at[pl.ds(pl.multiple_of(sid * ZROWS, 8), ZROWS)])
        pltpu.sync_copy(dst3_hbm.at[wid], idxv)
        pltpu.sync_copy(zero_hbm.at[pl.ds(0, SCHUNK)], rows)
        plsc.subcore_barrier()
        lane16 = lax.iota(jnp.int32, 16)
        jvecs = [jnp.full((16,), j, jnp.int32) for j in range(NCOL)]

        def chunk(c, carry):
            r_off = pl.multiple_of(base + c * SCHUNK, 8)
            descs = [pltpu.async_copy(col_hbm[j].at[pl.ds(r_off, SCHUNK)],
                                      cols.at[j], sem)
                     for j in range(NCOL)]
            for d in descs:
                d.wait()

            def interleave(g, carry2):
                ridx = g * 16 + lane16
                for j in range(NCOL):
                    vals = cols[j, pl.ds(g * 16, 16)]
                    plsc.store_scatter(rows, [ridx, jvecs[j]], vals)
                return carry2

            lax.fori_loop(0, SCHUNK // 16, interleave, None)
            for k in range(NSTR):
                pltpu.sync_copy(rows.at[pl.ds(k * 128, 128)],
                                acc.at[idxv.at[c * NSTR + k]], add=True)
            return carry

        lax.fori_loop(0, NSC, chunk, None)
        plsc.subcore_barrier()
        a_off = pl.multiple_of(sid * OROWS, 8)
        o_off = pl.multiple_of(cid * N + sid * OROWS, 8)

        @pl.when(sid < 15)
        def _():
            pltpu.sync_copy(acc.at[pl.ds(a_off, OROWS)],
                            out_hbm.at[pl.ds(o_off, OROWS)])

        @pl.when(sid == 15)
        def _():
            pltpu.sync_copy(acc.at[pl.ds(a_off, N - 15 * OROWS)],
                            out_hbm.at[pl.ds(o_off, N - 15 * OROWS)])

    return scatter_kernel


def _mlp_body(p0_ref, p1_ref, w1_ref, b1_ref, w2_ref, b2_ref,
              w3_ref, b3_ref, o_ref):
    s = p0_ref[...] + p1_ref[...]
    den = s[:, 0:1]
    den = jnp.where(den == 0.0, 1.0, den)
    h = s[:, 1:22] / den
    h = jnp.maximum(jnp.dot(h, w1_ref[...]) + b1_ref[...], 0.0)
    h = jnp.maximum(jnp.dot(h, w2_ref[...]) + b2_ref[...], 0.0)
    h = jnp.maximum(jnp.dot(h, w3_ref[...]) + b3_ref[...], 0.0)
    o_ref[...] = h


def _mlp_call(partials, w1t, b1r, w2t, b2r, w3t, b3r):
    full = lambda a: pl.BlockSpec(a.shape, lambda i: tuple(0 for _ in a.shape))
    return pl.pallas_call(
        _mlp_body,
        grid=(NBN,),
        in_specs=[
            pl.BlockSpec((BN, 24), lambda i: (i, 0)),
            pl.BlockSpec((BN, 24), lambda i: (i + NBN, 0)),
            full(w1t), full(b1r), full(w2t), full(b2r), full(w3t), full(b3r),
        ],
        out_specs=pl.BlockSpec((BN, 32), lambda i: (i, 0)),
        out_shape=jax.ShapeDtypeStruct((N, 32), jnp.float32),
    )(partials, partials, w1t, b1r, w2t, b2r, w3t, b3r)


def kernel(X_msg, edge_feat3, edge_lane, edge_src, edge_dst,
           TE_w, TE_lam, shared_W, shared_b,
           W_att, b_att, va, W1, b1, W2, b2, W3, b3):
    src_i = edge_src.astype(jnp.int32)
    dst_i = edge_dst.astype(jnp.int32)
    xflat = X_msg.reshape(-1)

    s0, s1, d0, d1 = _make_gather()(xflat, src_i, dst_i)

    e3 = lambda a: a.reshape(1, 1, E)
    r_cols = _edge_call(
        e3(edge_feat3[:, 0]), e3(edge_feat3[:, 1]), e3(edge_feat3[:, 2]),
        e3(edge_lane.astype(jnp.int32)),
        e3(s0), e3(s1), e3(d0), e3(d1),
        TE_w, TE_lam,
        shared_W, shared_b.reshape(8, 1),
        W_att, b_att.reshape(32, 1), va,
    )

    dst3 = jnp.pad(dst_i, (0, E_R - E),
                   constant_values=N_ACC - 1).reshape(NT, PS // 128, 128)
    partials = _make_scatter()(
        *[c.reshape(E_R) for c in r_cols],
        dst3, jnp.zeros((ZROWS, 24), jnp.float32))

    return _mlp_call(
        partials,
        W1.T, b1.reshape(1, 32),
        W2.T, b2.reshape(1, 32),
        W3.T, b3.reshape(1, 32),
    )


# speedup vs baseline: 1.6284x; 1.0211x over previous
import functools
import jax
import jax.numpy as jnp
from jax import lax
from jax.experimental import pallas as pl
from jax.experimental.pallas import tpu as pltpu
from jax.experimental.pallas import tpu_sc as plsc

N = 50000
E = 800000
NT = 32

PG = E // NT
CG = 1024
NCG = 24
TAILG = PG - NCG * CG

BE = 6400
NBE = E // BE

E_R = 802816
PS = E_R // NT
NCOL = 22
SCHUNK = 512
NSC = PS // SCHUNK
NSTR = SCHUNK // 128
N_ACC = 50048
ZROWS = 3128
OROWS = 3128

BN = 2000
NBN = N // BN


@functools.cache
def _make_gather():
    mesh = plsc.VectorSubcoreMesh(core_axis_name="c", subcore_axis_name="s")

    @functools.partial(
        pl.kernel,
        out_type=[jax.ShapeDtypeStruct((E,), jnp.float32) for _ in range(4)],
        mesh=mesh,
        compiler_params=pltpu.CompilerParams(needs_layout_passes=False),
        scratch_types=[
            pltpu.VMEM((2 * N,), jnp.float32),
            pltpu.VMEM((CG,), jnp.int32),
            pltpu.VMEM((CG,), jnp.int32),
            pltpu.VMEM((CG,), jnp.float32),
            pltpu.VMEM((CG,), jnp.float32),
            pltpu.VMEM((CG,), jnp.float32),
            pltpu.VMEM((CG,), jnp.float32),
            pltpu.SemaphoreType.DMA,
        ],
    )
    def gather_kernel(xflat_hbm, src_hbm, dst_hbm,
                      o_s0, o_s1, o_d0, o_d1,
                      xv, isv, idv, b_s0, b_s1, b_d0, b_d1, sem):
        wid = lax.axis_index("s") * 2 + lax.axis_index("c")
        base = wid * PG
        pltpu.sync_copy(xflat_hbm, xv)

        def gather16(i, clamp):
            s = isv[pl.ds(i * 16, 16)]
            d = idv[pl.ds(i * 16, 16)]
            if clamp:
                s = jnp.clip(s, 0, N - 1)
                d = jnp.clip(d, 0, N - 1)
            s = s * 2
            d = d * 2
            b_s0[pl.ds(i * 16, 16)] = plsc.load_gather(xv, [s])
            b_s1[pl.ds(i * 16, 16)] = plsc.load_gather(xv, [s + 1])
            b_d0[pl.ds(i * 16, 16)] = plsc.load_gather(xv, [d])
            b_d1[pl.ds(i * 16, 16)] = plsc.load_gather(xv, [d + 1])

        def run_chunk(off, n_edges, n_full16, tail16):
            ds = [pltpu.async_copy(src_hbm.at[pl.ds(off, n_edges)],
                                   isv.at[pl.ds(0, n_edges)], sem),
                  pltpu.async_copy(dst_hbm.at[pl.ds(off, n_edges)],
                                   idv.at[pl.ds(0, n_edges)], sem)]
            for d in ds:
                d.wait()

            def body(i, carry):
                gather16(i, False)
                return carry

            lax.fori_loop(0, n_full16, body, None)
            if tail16:
                gather16(n_full16, True)
            ds = [pltpu.async_copy(b_s0.at[pl.ds(0, n_edges)],
                                   o_s0.at[pl.ds(off, n_edges)], sem),
                  pltpu.async_copy(b_s1.at[pl.ds(0, n_edges)],
                                   o_s1.at[pl.ds(off, n_edges)], sem),
                  pltpu.async_copy(b_d0.at[pl.ds(0, n_edges)],
                                   o_d0.at[pl.ds(off, n_edges)], sem),
                  pltpu.async_copy(b_d1.at[pl.ds(0, n_edges)],
                                   o_d1.at[pl.ds(off, n_edges)], sem)]
            for d in ds:
                d.wait()

        for c in range(NCG):
            run_chunk(base + c * CG, CG, CG // 16, False)
        run_chunk(base + NCG * CG, TAILG, TAILG // 16, True)

    return gather_kernel


def _edge_body(c0_ref, dt_ref, c2_ref, lane_ref, s0_ref, s1_ref, d0_ref,
               d1_ref, tew_ref, telam_ref, sw_ref, sb_ref, wat_ref,
               batt_ref, va_ref, *o_refs):
    row = lambda ref: ref[0, 0, :].reshape(1, BE)
    c0 = row(c0_ref)
    dt = row(dt_ref)
    c2 = row(c2_ref)
    lane = row(lane_ref)
    oh = (lane == lax.broadcasted_iota(jnp.int32, (8, BE), 0)
          ).astype(jnp.float32)
    cdot = lambda a, b: lax.dot_general(a, b, (((0,), (0,)), ((), ())))
    ret_w = cdot(tew_ref[...], oh)
    ret_lam = cdot(telam_ref[...], oh)
    arg = dt * ret_w
    sh = sw_ref[...] * dt + sb_ref[...]
    lam = jnp.exp(-jnp.square(ret_lam))
    te_s = (1.0 - lam) * jnp.sin(arg) + lam * jnp.sin(sh)
    te_c = (1.0 - lam) * jnp.cos(arg) + lam * jnp.cos(sh)
    s0 = row(s0_ref)
    s1 = row(s1_ref)
    d0 = row(d0_ref)
    d1 = row(d1_ref)
    wat = wat_ref[...]
    col = lambda j: wat[:, j:j + 1]
    rdot = lambda a, b: lax.dot_general(a, b, (((1,), (0,)), ((), ())))
    z = (col(0) * s0 + col(1) * s1 + col(2) * d0 + col(3) * d1
         + col(4) * c0 + col(5) * dt + col(6) * c2
         + rdot(wat[:, 7:15], te_s)
         + rdot(wat[:, 15:23], te_c)
         + batt_ref[...])
    va = va_ref[...]
    att = rdot(va, jnp.tanh(z))
    bound = jnp.sum(jnp.abs(va))
    ex = jnp.exp(att - bound)
    es = ex * te_s
    ec = ex * te_c
    cols = [ex, ex * s0, ex * s1, ex * c0, ex * dt, ex * c2]
    cols += [es[j:j + 1, :] for j in range(8)]
    cols += [ec[j:j + 1, :] for j in range(8)]
    for j in range(NCOL):
        o_refs[j][0, 0, :] = cols[j].reshape(BE)


def _edge_call(c0, dtc, c2, lane3, s03, s13, d03, d13,
               tew, telam, sw2, sb2, wat, batt2, va):
    flat3 = pl.BlockSpec((1, 1, BE), lambda i: (0, 0, i))
    full = lambda a: pl.BlockSpec(a.shape, lambda i: tuple(0 for _ in a.shape))
    return pl.pallas_call(
        _edge_body,
        grid=(NBE,),
        in_specs=[
            flat3, flat3, flat3, flat3, flat3, flat3, flat3, flat3,
            full(tew), full(telam), full(sw2), full(sb2),
            full(wat), full(batt2), full(va),
        ],
        out_specs=[pl.BlockSpec((1, 1, BE), lambda i: (0, 0, i))
                   for _ in range(NCOL)],
        out_shape=[jax.ShapeDtypeStruct((1, 1, E_R), jnp.float32)
                   for _ in range(NCOL)],
    )(c0, dtc, c2, lane3, s03, s13, d03, d13,
      tew, telam, sw2, sb2, wat, batt2, va)


@functools.cache
def _make_scatter():
    mesh = plsc.VectorSubcoreMesh(core_axis_name="c", subcore_axis_name="s")

    @functools.partial(
        pl.kernel,
        out_type=jax.ShapeDtypeStruct((2 * N, 24), jnp.float32),
        mesh=mesh,
        compiler_params=pltpu.CompilerParams(
            needs_layout_passes=False, use_tc_tiling_on_sc=False),
        scratch_types=[
            pltpu.VMEM_SHARED((N_ACC, 24), jnp.float32),
            pltpu.VMEM((PS // 128, 128), jnp.int32),
            pltpu.VMEM((NCOL, SCHUNK), jnp.float32),
            pltpu.VMEM((SCHUNK, 24), jnp.float32),
            pltpu.SemaphoreType.DMA,
        ],
    )
    def scatter_kernel(*args):
        col_hbm = args[:NCOL]
        dst3_hbm, zero_hbm, out_hbm, acc, idxv, cols, rows, sem = args[NCOL:]
        cid = lax.axis_index("c")
        sid = lax.axis_index("s")
        wid = sid * 2 + cid
        base = wid * PS
        pltpu.sync_copy(zero_hbm,
                        acc.at[pl.ds(pl.multiple_of(sid * ZROWS, 8), ZROWS)])
        pltpu.sync_copy(dst3_hbm.at[wid], idxv)
        pltpu.sync_copy(zero_hbm.at[pl.ds(0, SCHUNK)], rows)
        plsc.subcore_barrier()
        lane16 = lax.iota(jnp.int32, 16)
        jvecs = [jnp.full((16,), j, jnp.int32) for j in range(NCOL)]

        def chunk(c, carry):
            r_off = pl.multiple_of(base + c * SCHUNK, 8)
            descs = [pltpu.async_copy(col_hbm[j].at[pl.ds(r_off, SCHUNK)],
                                      cols.at[j], sem)
                     for j in range(NCOL)]
            for d in descs:
                d.wait()

            def interleave(g, carry2):
                ridx = g * 16 + lane16
                for j in range(NCOL):
                    vals = cols[j, pl.ds(g * 16, 16)]
                    plsc.store_scatter(rows, [ridx, jvecs[j]], vals)
                return carry2

            lax.fori_loop(0, SCHUNK // 16, interleave, None)
            for k in range(NSTR):
                pltpu.sync_copy(rows.at[pl.ds(k * 128, 128)],
                                acc.at[idxv.at[c * NSTR + k]], add=True)
            return carry

        lax.fori_loop(0, NSC, chunk, None)
        plsc.subcore_barrier()
        a_off = pl.multiple_of(sid * OROWS, 8)
        o_off = pl.multiple_of(cid * N + sid * OROWS, 8)

        @pl.when(sid < 15)
        def _():
            pltpu.sync_copy(acc.at[pl.ds(a_off, OROWS)],
                            out_hbm.at[pl.ds(o_off, OROWS)])

        @pl.when(sid == 15)
        def _():
            pltpu.sync_copy(acc.at[pl.ds(a_off, N - 15 * OROWS)],
                            out_hbm.at[pl.ds(o_off, N - 15 * OROWS)])

    return scatter_kernel


def _mlp_body(p0_ref, p1_ref, w1_ref, b1_ref, w2_ref, b2_ref,
              w3_ref, b3_ref, o_ref):
    s = p0_ref[...] + p1_ref[...]
    den = s[:, 0:1]
    den = jnp.where(den == 0.0, 1.0, den)
    h = s[:, 1:22] / den
    h = jnp.maximum(jnp.dot(h, w1_ref[...]) + b1_ref[...], 0.0)
    h = jnp.maximum(jnp.dot(h, w2_ref[...]) + b2_ref[...], 0.0)
    h = jnp.maximum(jnp.dot(h, w3_ref[...]) + b3_ref[...], 0.0)
    o_ref[...] = h


def _mlp_call(partials, w1t, b1r, w2t, b2r, w3t, b3r):
    full = lambda a: pl.BlockSpec(a.shape, lambda i: tuple(0 for _ in a.shape))
    return pl.pallas_call(
        _mlp_body,
        grid=(NBN,),
        in_specs=[
            pl.BlockSpec((BN, 24), lambda i: (i, 0)),
            pl.BlockSpec((BN, 24), lambda i: (i + NBN, 0)),
            full(w1t), full(b1r), full(w2t), full(b2r), full(w3t), full(b3r),
        ],
        out_specs=pl.BlockSpec((BN, 32), lambda i: (i, 0)),
        out_shape=jax.ShapeDtypeStruct((N, 32), jnp.float32),
    )(partials, partials, w1t, b1r, w2t, b2r, w3t, b3r)


def kernel(X_msg, edge_feat3, edge_lane, edge_src, edge_dst,
           TE_w, TE_lam, shared_W, shared_b,
           W_att, b_att, va, W1, b1, W2, b2, W3, b3):
    src_i = edge_src.astype(jnp.int32)
    dst_i = edge_dst.astype(jnp.int32)
    xflat = X_msg.reshape(-1)

    s0, s1, d0, d1 = _make_gather()(xflat, src_i, dst_i)

    e3 = lambda a: a.reshape(1, 1, E)
    r_cols = _edge_call(
        e3(edge_feat3[:, 0]), e3(edge_feat3[:, 1]), e3(edge_feat3[:, 2]),
        e3(edge_lane.astype(jnp.int32)),
        e3(s0), e3(s1), e3(d0), e3(d1),
        TE_w, TE_lam,
        shared_W, shared_b.reshape(8, 1),
        W_att, b_att.reshape(32, 1), va,
    )

    dst3 = jnp.pad(dst_i, (0, E_R - E),
                   constant_values=N_ACC - 1).reshape(NT, PS // 128, 128)
    partials = _make_scatter()(
        *[c.reshape(E_R) for c in r_cols],
        dst3, jnp.zeros((ZROWS, 24), jnp.float32))

    return _mlp_call(
        partials,
        W1.T, b1.reshape(1, 32),
        W2.T, b2.reshape(1, 32),
        W3.T, b3.reshape(1, 32),
    )
